# Initial kernel scaffold; baseline (speedup 1.0000x reference)
import jax
import jax.numpy as jnp
from jax.experimental import pallas as pl


def kernel(v_attr, faces_v_idx, face_index_map, weight_map):
    raise NotImplementedError("write your pallas kernel here")


# speedup vs baseline: 68.0288x; 68.0288x over previous
import functools

import jax
import jax.numpy as jnp
from jax import lax
from jax.experimental import pallas as pl
from jax.experimental.pallas import tpu as pltpu
from jax.experimental.pallas import tpu_sc as plsc

_NW = 32
_C = 1024
_IB = 128
_NJ = _C // _IB


def _sc_body(fim_hbm, w_hbm, fv0_hbm, fv1_hbm, fv2_hbm, va_hbm, out_hbm,
             fim_v, fvd0, fvd1, fvd2, a0, a1, a2, wv, ov, sem):
    n_chunks = fim_hbm.shape[0] // _NJ // _NW
    pix_per_w = n_chunks * _C
    wid = lax.axis_index("s") * 2 + lax.axis_index("c")
    batch_pix = 512 * 512
    off = (wid * pix_per_w // batch_pix) * jnp.int32(200000)

    def chunk(t, carry):
        base_row = wid * (n_chunks * _NJ) + t * _NJ
        base_pix = base_row * _IB
        pltpu.sync_copy(fim_hbm.at[pl.ds(base_row, _NJ)], fim_v)
        pltpu.sync_copy(w_hbm.at[:, pl.ds(base_pix, _C)], wv)
        for i in range(_C // 16):
            r, c = i // (_IB // 16), (i % (_IB // 16)) * 16
            fim_v[r, pl.ds(c, 16)] = fim_v[r, pl.ds(c, 16)] + off
        cps = []
        for j in range(_NJ):
            cps.append(pltpu.async_copy(fv0_hbm.at[fim_v.at[j]], fvd0.at[j], sem))
            cps.append(pltpu.async_copy(fv1_hbm.at[fim_v.at[j]], fvd1.at[j], sem))
            cps.append(pltpu.async_copy(fv2_hbm.at[fim_v.at[j]], fvd2.at[j], sem))
        for cp in cps:
            cp.wait()
        cps = []
        for j in range(_NJ):
            cps.append(pltpu.async_copy(va_hbm.at[fvd0.at[j]], a0.at[pl.ds(j * _IB, _IB)], sem))
            cps.append(pltpu.async_copy(va_hbm.at[fvd1.at[j]], a1.at[pl.ds(j * _IB, _IB)], sem))
            cps.append(pltpu.async_copy(va_hbm.at[fvd2.at[j]], a2.at[pl.ds(j * _IB, _IB)], sem))
        for cp in cps:
            cp.wait()

        def grp(g, c2):
            p0 = g * 16
            w0v = wv[0, pl.ds(p0, 16)]
            w1v = wv[1, pl.ds(p0, 16)]
            w2v = wv[2, pl.ds(p0, 16)]
            for q in range(16):
                p = p0 + q
                acc = (a0[p, :] * w0v[q] + a1[p, :] * w1v[q]
                       + a2[p, :] * w2v[q])
                ov[p, :] = acc
            return c2

        lax.fori_loop(0, _C // 16, grp, 0)
        pltpu.sync_copy(ov, out_hbm.at[pl.ds(base_pix, _C)])
        return carry

    lax.fori_loop(0, n_chunks, chunk, 0)


def kernel(v_attr, faces_v_idx, face_index_map, weight_map):
    N, H, W = face_index_map.shape
    V, A = v_attr.shape[1], v_attr.shape[2]
    F = faces_v_idx.shape[1]
    P = N * H * W

    fim = face_index_map.astype(jnp.int32).reshape(P // _IB, _IB)
    fv = faces_v_idx.astype(jnp.int32).reshape(N * F, 3)
    fv0, fv1, fv2 = fv[:, 0], fv[:, 1], fv[:, 2]
    wm = weight_map.reshape(P, 3).T
    va = v_attr.reshape(V, A)

    mesh = plsc.VectorSubcoreMesh(core_axis_name="c", subcore_axis_name="s")
    run = pl.kernel(
        _sc_body,
        mesh=mesh,
        compiler_params=pltpu.CompilerParams(use_tc_tiling_on_sc=False),
        out_type=jax.ShapeDtypeStruct((P, A), jnp.float32),
        scratch_types=[
            pltpu.VMEM((_NJ, _IB), jnp.int32),
            pltpu.VMEM((_NJ, _IB), jnp.int32),
            pltpu.VMEM((_NJ, _IB), jnp.int32),
            pltpu.VMEM((_NJ, _IB), jnp.int32),
            pltpu.VMEM((_C, 16), jnp.float32),
            pltpu.VMEM((_C, 16), jnp.float32),
            pltpu.VMEM((_C, 16), jnp.float32),
            pltpu.VMEM((3, _C), jnp.float32),
            pltpu.VMEM((_C, 16), jnp.float32),
            pltpu.SemaphoreType.DMA,
        ],
    )
    out = run(fim, wm, fv0, fv1, fv2, va)
    return out.reshape(N, H, W, A)
